# baseline (device time: 8254 ns/iter reference)
import jax
import jax.numpy as jnp
from jax import lax
from jax.experimental import pallas as pl
from jax.experimental.pallas import tpu as pltpu

N_CHUNK = 4


def kernel(x, dy, gamma):
    m, d = x.shape
    half_m = m // 2
    rows = half_m // N_CHUNK
    f32 = jnp.float32
    bf16 = jnp.bfloat16

    def body(x_hbm, dy_hbm, out_ref, xv_ref, dyv_ref, acc_ref, comm_ref,
             xsems, dysems, send_sems, recv_sems, credit_sems):
        my_x = lax.axis_index("x")
        my_y = lax.axis_index("y")
        my_z = lax.axis_index("z")
        peers = [
            (1 - my_x, my_y, my_z),
            (my_x, my_y, 1 - my_z),
            (1 - my_x, my_y, 1 - my_z),
        ]

        base = my_x * half_m
        dmas = []
        for c in range(N_CHUNK):
            src_sl = pl.ds(base + c * rows, rows)
            dst_sl = pl.ds(c * rows, rows)
            dx = pltpu.make_async_copy(
                x_hbm.at[src_sl, :], xv_ref.at[dst_sl, :], xsems.at[c])
            ddy = pltpu.make_async_copy(
                dy_hbm.at[src_sl, :], dyv_ref.at[dst_sl, :], dysems.at[c])
            dx.start()
            ddy.start()
            dmas.append((dx, ddy))

        barrier_sem = pltpu.get_barrier_semaphore()
        for dir_, peer in enumerate(peers):
            pl.semaphore_signal(
                barrier_sem, inc=1, device_id=peer,
                device_id_type=pl.DeviceIdType.MESH,
            )
            pl.semaphore_signal(
                credit_sems.at[dir_], inc=1, device_id=peer,
                device_id_type=pl.DeviceIdType.MESH,
            )

        for c in range(N_CHUNK):
            dx, ddy = dmas[c]
            dx.wait()
            ddy.wait()
            sl = pl.ds(c * rows, rows)
            xb = xv_ref[sl, :].astype(bf16)
            dyb = dyv_ref[sl, :].astype(bf16)
            mu = jnp.mean(xb, axis=1, keepdims=True, dtype=f32)
            var = jnp.mean(xb * xb, axis=1, keepdims=True, dtype=f32) - mu * mu
            rstd = lax.rsqrt(var + 1e-5)
            xhat = (xb - mu.astype(bf16)) * rstd.astype(bf16)
            dgamma = jnp.sum(dyb * xhat, axis=0, keepdims=True, dtype=f32)
            dbeta = jnp.sum(dyb, axis=0, keepdims=True, dtype=f32)
            partial = jnp.concatenate([dgamma, dbeta], axis=0)
            if c == 0:
                acc_ref[...] = partial
            else:
                acc_ref[...] = acc_ref[...] + partial

        comm_ref[0] = acc_ref[...]

        pl.semaphore_wait(barrier_sem, 3)

        rdmas = []
        for dir_, peer in enumerate(peers):
            pl.semaphore_wait(credit_sems.at[dir_], 1)
            rdma = pltpu.make_async_remote_copy(
                src_ref=comm_ref.at[0],
                dst_ref=comm_ref.at[1 + dir_],
                send_sem=send_sems.at[dir_],
                recv_sem=recv_sems.at[dir_],
                device_id=peer,
                device_id_type=pl.DeviceIdType.MESH,
            )
            rdma.start()
            rdmas.append(rdma)
        for rdma in rdmas:
            rdma.wait()

        out_ref[...] = (comm_ref[0] + comm_ref[1]) + (comm_ref[2] + comm_ref[3])

    return pl.pallas_call(
        body,
        out_shape=jax.ShapeDtypeStruct((2, d), jnp.float32),
        in_specs=[
            pl.BlockSpec(memory_space=pltpu.MemorySpace.HBM),
            pl.BlockSpec(memory_space=pltpu.MemorySpace.HBM),
        ],
        out_specs=pl.BlockSpec(memory_space=pltpu.VMEM),
        scratch_shapes=[
            pltpu.VMEM((half_m, d), f32),
            pltpu.VMEM((half_m, d), f32),
            pltpu.VMEM((2, d), f32),
            pltpu.VMEM((4, 2, d), f32),
            pltpu.SemaphoreType.DMA((N_CHUNK,)),
            pltpu.SemaphoreType.DMA((N_CHUNK,)),
            pltpu.SemaphoreType.DMA((3,)),
            pltpu.SemaphoreType.DMA((3,)),
            pltpu.SemaphoreType.REGULAR((3,)),
        ],
        compiler_params=pltpu.CompilerParams(collective_id=0),
    )(
        pltpu.with_memory_space_constraint(x, pltpu.MemorySpace.HBM),
        pltpu.with_memory_space_constraint(dy, pltpu.MemorySpace.HBM),
    )
